# baseline (speedup 1.0000x reference)
import functools

import jax
import jax.numpy as jnp
from jax import lax
from jax.experimental import pallas as pl
from jax.experimental.pallas import tpu as pltpu
from jax.experimental.pallas import tpu_sc as plsc

_N = 10000
_E = 320000
_NIN = 128
_NHID = 128
_NOUT = 10
_VAR_EPS = 1e-4
_L = 16
_NC = 2
_NS = 16
_NW = _NC * _NS
_CB = 80
_NCHUNK = 125
_NPAIR = (_NCHUNK - 1) // 2
_EPT = _CB * _NCHUNK
_RPT = 640
_SPAD = _RPT * _NS
_NPAD = 16000
_HP = lax.Precision.HIGHEST

import numpy as _np


def _tf_rounds(x0, x1, rots):
    for r in rots:
        x0 = (x0 + x1).astype(_np.uint32)
        x1 = ((x1 << _np.uint32(r)) | (x1 >> _np.uint32(32 - r))).astype(
            _np.uint32)
        x1 = x1 ^ x0
    return x0, x1


def _threefry2x32(k1, k2, x0, x1):
    r1 = (13, 15, 26, 6)
    r2 = (17, 29, 16, 24)
    ks0 = _np.uint32(k1)
    ks1 = _np.uint32(k2)
    ks2 = ks0 ^ ks1 ^ _np.uint32(0x1BD11BDA)
    u = _np.uint32
    x0 = (x0 + ks0).astype(u)
    x1 = (x1 + ks1).astype(u)
    for i, (ka, kb, rr) in enumerate(((ks1, ks2, r1), (ks2, ks0, r2),
                                      (ks0, ks1, r1), (ks1, ks2, r2),
                                      (ks2, ks0, r1))):
        x0, x1 = _tf_rounds(x0, x1, rr)
        x0 = (x0 + ka).astype(u)
        x1 = (x1 + kb + u(i + 1)).astype(u)
    return x0, x1


def _erfinv32(u):
    w = -_np.log((1.0 - u.astype(_np.float64)) * (1.0 + u))
    p = _np.where(
        w < 5.0,
        _np.polyval([2.81022636e-08, 3.43273939e-07, -3.5233877e-06,
                     -4.39150654e-06, 0.00021858087, -0.00125372503,
                     -0.00417768164, 0.246640727, 1.50140941], w - 2.5),
        _np.polyval([-0.000200214257, 0.000100950558, 0.00134934322,
                     -0.00367342844, 0.00573950773, -0.0076224613,
                     0.00943887047, 1.00167406, 2.83297682],
                    _np.sqrt(_np.maximum(w, 5.0)) - 3.0))
    return (p * u).astype(_np.float32)


def _const_normal(seed):
    cnt = _np.arange(_N * _NOUT, dtype=_np.uint32)
    b1, b2 = _threefry2x32(0, seed, _np.zeros_like(cnt), cnt)
    bits = b1 ^ b2
    fb = (bits >> _np.uint32(9)) | _np.float32(1.0).view(_np.uint32)
    floats = fb.view(_np.float32) - _np.float32(1.0)
    lo = _np.float32(_np.nextafter(_np.float32(-1.0), _np.float32(0.0)))
    hi = _np.float32(1.0)
    uni = _np.maximum(lo, (floats * (hi - lo) + lo).astype(_np.float32))
    vals = (_np.float32(_np.sqrt(2.0)) * _erfinv32(uni)).astype(_np.float32)
    return _np.pad(vals.reshape(_N, _NOUT), ((0, 0), (0, _L - _NOUT)))


_EPS1P = _const_normal(1)
_EPS2P = _const_normal(2)


_GD = lax.GatherDimensionNumbers(
    offset_dims=(), collapsed_slice_dims=(0,), start_index_map=(0,))


def _lanesum(t):
    lanes = lax.iota(jnp.int32, _L)
    for k in (8, 4, 2, 1):
        t = t + lax.gather(t, (lanes ^ k)[:, None], _GD, slice_sizes=(1,),
                           mode=lax.GatherScatterMode.PROMISE_IN_BOUNDS)
    return t


def _bn(h, g, b):
    m = jnp.mean(h, axis=0, keepdims=True)
    v = jnp.mean((h - m) ** 2, axis=0, keepdims=True)
    return (h - m) * lax.rsqrt(v + 1e-5) * g + b


_BLK = 2000
_G = _N // _BLK


def _rowspec(width):
    return pl.BlockSpec((_BLK, width), lambda i: (i, 0))


def _fullspec(shape):
    return pl.BlockSpec(shape, lambda i: (0, 0))


def _stats(st_ref):
    m = st_ref[0:1, :] * (1.0 / _N)
    v = st_ref[1:2, :] * (1.0 / _N) - m * m
    return m, lax.rsqrt(v + 1e-5)


def _enc_a_body(x_ref, bif_ref, w0x_ref, w0c_ref, p0_ref,
                hpre_ref, rsum_ref, st_ref, acc):
    i = pl.program_id(0)

    @pl.when(i == 0)
    def _():
        acc[...] = jnp.zeros_like(acc)

    x = x_ref[...]
    xl = jnp.log1p(x)
    sel = jnp.where(bif_ref[...] == 0, w0c_ref[0:1, :], w0c_ref[1:2, :])
    hp = jnp.dot(xl, w0x_ref[...], precision=_HP) + sel + p0_ref[0:1, :]
    hpre_ref[...] = hp
    rsum_ref[...] = jnp.broadcast_to(
        jnp.sum(x, axis=1, keepdims=True), (_BLK, _L))
    acc[0:1, :] = acc[0:1, :] + jnp.sum(hp, axis=0, keepdims=True)
    acc[1:2, :] = acc[1:2, :] + jnp.sum(hp * hp, axis=0, keepdims=True)

    @pl.when(i == _G - 1)
    def _():
        st_ref[...] = acc[...]


def _enc_b_body(hpre_ref, bif_ref, st0_ref, p0_ref, w1x_ref, w1c_ref, p1_ref,
                qpre_ref, st_ref, acc):
    i = pl.program_id(0)

    @pl.when(i == 0)
    def _():
        acc[...] = jnp.zeros_like(acc)

    m0, r0 = _stats(st0_ref)
    h = jax.nn.relu((hpre_ref[...] - m0) * r0 * p0_ref[1:2, :]
                    + p0_ref[2:3, :])
    sel = jnp.where(bif_ref[...] == 0, w1c_ref[0:1, :], w1c_ref[1:2, :])
    qp = jnp.dot(h, w1x_ref[...], precision=_HP) + sel + p1_ref[0:1, :]
    qpre_ref[...] = qp
    acc[0:1, :] = acc[0:1, :] + jnp.sum(qp, axis=0, keepdims=True)
    acc[1:2, :] = acc[1:2, :] + jnp.sum(qp * qp, axis=0, keepdims=True)

    @pl.when(i == _G - 1)
    def _():
        st_ref[...] = acc[...]


def _enc_c_body(qpre_ref, st1_ref, p1_ref, whead_ref, hb_ref, attab_ref,
                eps1_ref, tabs_ref, z_ref, attp_ref, acc):
    i = pl.program_id(0)

    @pl.when(i == 0)
    def _():
        acc[...] = jnp.zeros_like(acc)

    m1, r1 = _stats(st1_ref)
    q = jax.nn.relu((qpre_ref[...] - m1) * r1 * p1_ref[1:2, :]
                    + p1_ref[2:3, :])
    heads = jnp.dot(q, whead_ref[...], precision=_HP)
    col = lax.broadcasted_iota(jnp.int32, (1, 128), 1)
    t4 = jnp.concatenate(
        [heads[:, 32:96], jnp.zeros((_BLK, 64), jnp.float32)], axis=1)
    tabs_ref[...] = jnp.where((col == 10) | (col == 42), 1.0, t4)
    qm = heads[:, 0:16] + hb_ref[0:1, :]
    qv = jnp.exp(heads[:, 16:32] + hb_ref[1:2, :]) + _VAR_EPS
    z_ref[...] = qm + jnp.sqrt(qv) * eps1_ref[...]
    acc[0:1, :] = jnp.maximum(
        acc[0:1, :], jnp.max(jnp.abs(heads), axis=0, keepdims=True))

    @pl.when(i == _G - 1)
    def _():
        cm = acc[0:1, :]
        am = jnp.abs(attab_ref[0:1, :])
        av = jnp.abs(attab_ref[1:2, :])
        bm = jnp.sum((cm[:, 32:48] + cm[:, 48:64]) * am)
        bv = jnp.sum((cm[:, 64:80] + cm[:, 80:96]) * av)
        attw = jnp.concatenate(
            [attab_ref[...], jnp.zeros((2, 112), jnp.float32)], axis=1)
        rowid = lax.broadcasted_iota(jnp.int32, (8, 128), 0)
        bnds = jnp.where(rowid < 3, bm, bv)
        attp_ref[...] = jnp.where(
            rowid == 0, attw[0:1, :],
            jnp.where(rowid == 1, attw[1:2, :],
                      jnp.where(rowid < 4, bnds, 0.0)))


def _encode(x, bif, w0x, w0c, p0, w1x, w1c, p1, whead, hb, attab, eps1p):
    f32 = jnp.float32
    sds = jax.ShapeDtypeStruct
    hpre, rsum, st0 = pl.pallas_call(
        _enc_a_body,
        grid=(_G,),
        in_specs=[_rowspec(128), _rowspec(1), _fullspec((128, 128)),
                  _fullspec((2, 128)), _fullspec((3, 128))],
        out_specs=[_rowspec(128), _rowspec(_L), _fullspec((8, 128))],
        out_shape=[sds((_N, 128), f32), sds((_N, _L), f32),
                   sds((8, 128), f32)],
        scratch_shapes=[pltpu.VMEM((8, 128), f32)],
    )(x, bif, w0x, w0c, p0)
    qpre, st1 = pl.pallas_call(
        _enc_b_body,
        grid=(_G,),
        in_specs=[_rowspec(128), _rowspec(1), _fullspec((8, 128)),
                  _fullspec((3, 128)), _fullspec((128, 128)),
                  _fullspec((2, 128)), _fullspec((3, 128))],
        out_specs=[_rowspec(128), _fullspec((8, 128))],
        out_shape=[sds((_N, 128), f32), sds((8, 128), f32)],
        scratch_shapes=[pltpu.VMEM((8, 128), f32)],
    )(hpre, bif, st0, p0, w1x, w1c, p1)
    tabs, z, attp = pl.pallas_call(
        _enc_c_body,
        grid=(_G,),
        in_specs=[_rowspec(128), _fullspec((8, 128)), _fullspec((3, 128)),
                  _fullspec((128, 128)), _fullspec((2, 16)),
                  _fullspec((2, 16)), _rowspec(_L)],
        out_specs=[_rowspec(128), _rowspec(_L), _fullspec((8, 128))],
        out_shape=[sds((_SPAD, 128), f32), sds((_N, _L), f32),
                   sds((8, 128), f32)],
        scratch_shapes=[pltpu.VMEM((8, 128), f32)],
    )(qpre, st1, p1, whead, hb, attab, eps1p)
    return tabs, z, rsum, attp


def _gat_body(edge_hbm, xt_hbm, attp_hbm, out_hbm,
              src_v, dst_v, ra0, ra1, ra2, ra3, rb0, rb1, rb2, rb3,
              zbuf, attp_v, xlm_sh, xrm_sh, xlv_sh, xrv_sh, accm, accv, sem):
    c = lax.axis_index("c")
    s = lax.axis_index("s")
    wid = c * _NS + s
    pltpu.sync_copy(attp_hbm, attp_v)
    attm_v = attp_v[0, 0:16]
    attv_v = attp_v[1, 0:16]
    bm_v = attp_v[2, 0:16]
    bv_v = attp_v[3, 0:16]
    row0 = s * _RPT
    st = pl.ds(row0, _RPT)
    pltpu.sync_copy(xt_hbm.at[st, pl.ds(0, 16)], xlm_sh.at[st])
    pltpu.sync_copy(xt_hbm.at[st, pl.ds(16, 16)], xrm_sh.at[st])
    pltpu.sync_copy(xt_hbm.at[st, pl.ds(32, 16)], xlv_sh.at[st])
    pltpu.sync_copy(xt_hbm.at[st, pl.ds(48, 16)], xrv_sh.at[st])

    def _z(i, carry):
        zbuf[i, :] = jnp.zeros((_L,), jnp.float32)
        return carry

    lax.fori_loop(0, _RPT, _z, 0)
    pltpu.sync_copy(zbuf, accm.at[st])
    pltpu.sync_copy(zbuf, accv.at[st])
    pltpu.sync_copy(edge_hbm.at[0, wid], src_v)
    pltpu.sync_copy(edge_hbm.at[1, wid], dst_v)
    plsc.subcore_barrier()

    def _issue(j, rl_m, rr_m, rl_v, rr_v):
        sidx = src_v.at[j]
        didx = dst_v.at[j]
        pltpu.async_copy(xlm_sh.at[sidx], rl_m, sem)
        pltpu.async_copy(xrm_sh.at[didx], rr_m, sem)
        pltpu.async_copy(xlv_sh.at[sidx], rl_v, sem)
        pltpu.async_copy(xrv_sh.at[didx], rr_v, sem)

    def _drain(*bufs):
        for b in bufs:
            pltpu.make_async_copy(
                xt_hbm.at[pl.ds(0, _CB), pl.ds(0, 16)], b, sem).wait()

    def _compute(rl_m, rr_m, rl_v, rr_v):
        @plsc.parallel_loop(0, _CB, 1, unroll=16)
        def _(k):
            a_m = rl_m[k, :]
            b_m = rr_m[k, :]
            mm = a_m + b_m
            lrm = jnp.maximum(mm, mm * 0.2)
            eem = jnp.exp(_lanesum(lrm * attm_v) - bm_v)
            rl_m[k, :] = a_m * eem
            a_v = rl_v[k, :]
            b_v = rr_v[k, :]
            mv = a_v + b_v
            lrv = jnp.maximum(mv, mv * 0.2)
            eev = jnp.exp(_lanesum(lrv * attv_v) - bv_v)
            rl_v[k, :] = a_v * eev

    def _scatter(j, rl_m, rl_v):
        didx = dst_v.at[j]
        pltpu.sync_copy(rl_m, accm.at[didx], add=True)
        pltpu.sync_copy(rl_v, accv.at[didx], add=True)

    _issue(0, ra0, ra1, ra2, ra3)

    def _pair(p, carry):
        j0 = 2 * p
        _issue(j0 + 1, rb0, rb1, rb2, rb3)
        _drain(ra0, ra1, ra2, ra3)
        _compute(ra0, ra1, ra2, ra3)
        _scatter(j0, ra0, ra2)
        _issue(j0 + 2, ra0, ra1, ra2, ra3)
        _drain(rb0, rb1, rb2, rb3)
        _compute(rb0, rb1, rb2, rb3)
        _scatter(j0 + 1, rb0, rb2)
        return carry

    lax.fori_loop(0, _NPAIR, _pair, 0)
    _drain(ra0, ra1, ra2, ra3)
    _compute(ra0, ra1, ra2, ra3)
    _scatter(_NCHUNK - 1, ra0, ra2)
    plsc.subcore_barrier()
    out_row = c * _NPAD + row0
    pltpu.sync_copy(accm.at[st], out_hbm.at[pl.ds(out_row, _RPT), pl.ds(0, 16)])
    pltpu.sync_copy(accv.at[st], out_hbm.at[pl.ds(out_row, _RPT), pl.ds(16, 16)])


def _gat_call(edges, tabs, attp):
    f32 = jnp.float32
    i32 = jnp.int32
    run = functools.partial(
        pl.kernel,
        mesh=plsc.VectorSubcoreMesh(core_axis_name="c", subcore_axis_name="s"),
        compiler_params=pltpu.CompilerParams(use_tc_tiling_on_sc=False),
        out_type=jax.ShapeDtypeStruct((_NC * _NPAD, 128), f32),
        scratch_types=(
            [pltpu.VMEM((_NCHUNK, _CB), i32),
             pltpu.VMEM((_NCHUNK, _CB), i32)]
            + [pltpu.VMEM((_CB, _L), f32) for _ in range(8)]
            + [pltpu.VMEM((_RPT, _L), f32),
               pltpu.VMEM((8, 128), f32),
               pltpu.VMEM_SHARED((_SPAD, _L), f32),
               pltpu.VMEM_SHARED((_SPAD, _L), f32),
               pltpu.VMEM_SHARED((_SPAD, _L), f32),
               pltpu.VMEM_SHARED((_SPAD, _L), f32),
               pltpu.VMEM_SHARED((_SPAD, _L), f32),
               pltpu.VMEM_SHARED((_SPAD, _L), f32),
               pltpu.SemaphoreType.DMA]
        ),
    )(_gat_body)
    return run(edges, tabs, attp)


def _dec_a_body(lo_ref, hi_ref, z_ref, bif_ref, gb_ref,
                eps2_ref, wd0z_ref, wd0c_ref, pd_ref,
                hpre_ref, st_ref, acc):
    i = pl.program_id(0)

    @pl.when(i == 0)
    def _():
        acc[...] = jnp.zeros_like(acc)

    lo = lo_ref[...]
    hi = hi_ref[...]
    nm = lo[:, 0:16] + hi[:, 0:16]
    nv = lo[:, 16:32] + hi[:, 16:32]
    qm = nm / (nm[:, 10:11] + 1e-16) + gb_ref[0:1, :]
    qv = jnp.exp(nv / (nv[:, 10:11] + 1e-16) + gb_ref[1:2, :]) + _VAR_EPS
    zg = qm + jnp.sqrt(qv) * eps2_ref[...]
    zall = jnp.concatenate([zg, z_ref[...]], axis=1)
    seld = jnp.where(bif_ref[...] == 0, wd0c_ref[0:1, :], wd0c_ref[1:2, :])
    hp = jnp.dot(zall, wd0z_ref[...], precision=_HP) + seld + pd_ref[0:1, :]
    hpre_ref[...] = hp
    acc[0:1, :] = acc[0:1, :] + jnp.sum(hp, axis=0, keepdims=True)
    acc[1:2, :] = acc[1:2, :] + jnp.sum(hp * hp, axis=0, keepdims=True)

    @pl.when(i == _G - 1)
    def _():
        st_ref[...] = acc[...]


def _dec_b_body(hpre_ref, std_ref, pd_ref, wsc_ref, bsc_ref, wdr_ref,
                bdr_ref, rsum_ref, ps_ref, pr_ref, pdo_ref):
    md, rd = _stats(std_ref)
    hd = jax.nn.relu((hpre_ref[...] - md) * rd * pd_ref[1:2, :]
                     + pd_ref[2:3, :])
    psl = jnp.dot(hd, wsc_ref[...], precision=_HP) + bsc_ref[...]
    psl = psl - jnp.max(psl, axis=1, keepdims=True)
    epl = jnp.exp(psl)
    ps = epl / jnp.sum(epl, axis=1, keepdims=True)
    ps_ref[...] = ps
    pr_ref[...] = rsum_ref[:, 0:1] * ps
    pdo_ref[...] = jnp.dot(hd, wdr_ref[...], precision=_HP) + bdr_ref[...]


def _decode(out, z, rsum, bif, gb, eps2p, wd0z, wd0c, pd, wsc, bsc,
            wdr, bdr):
    f32 = jnp.float32
    sds = jax.ShapeDtypeStruct
    off = _NPAD // _BLK
    hispec = pl.BlockSpec((_BLK, 128), lambda i: (i + off, 0))
    hpre, std = pl.pallas_call(
        _dec_a_body,
        grid=(_G,),
        in_specs=[_rowspec(128), hispec,
                  _rowspec(_L), _rowspec(1), _fullspec((2, 16)),
                  _rowspec(_L), _fullspec((32, 128)), _fullspec((2, 128)),
                  _fullspec((3, 128))],
        out_specs=[_rowspec(128), _fullspec((8, 128))],
        out_shape=[sds((_N, 128), f32), sds((8, 128), f32)],
        scratch_shapes=[pltpu.VMEM((8, 128), f32)],
    )(out, out, z, bif, gb, eps2p, wd0z, wd0c, pd)
    return pl.pallas_call(
        _dec_b_body,
        grid=(_G,),
        in_specs=[_rowspec(128), _fullspec((8, 128)), _fullspec((3, 128)),
                  _fullspec((128, 128)), _fullspec((1, 128)),
                  _fullspec((128, 128)), _fullspec((1, 128)), _rowspec(_L)],
        out_specs=[_rowspec(128), _rowspec(128), _rowspec(128)],
        out_shape=[sds((_N, 128), f32), sds((_N, 128), f32),
                   sds((_N, 128), f32)],
    )(hpre, std, pd, wsc, bsc, wdr, bdr, rsum)


def kernel(x, batch_index, edge_index, W0, b0, g0, be0, W1, b1, g1, be1, Wm,
           bm, Wv, bv, Wlm, Wrm, attm, biasm, Wlv, Wrv, attv, biasv, Wd0, bd0,
           gd0, bed0, Wscale, bscale, Wdrop, bdrop, px_r):
    f32 = jnp.float32

    def padw(w):
        return jnp.pad(w.T, ((0, 0), (0, _L - _NOUT)))

    def padv(v):
        return jnp.pad(v, (0, _L - _NOUT)).reshape(1, _L)

    bif = batch_index
    w0x = W0[:, :_NIN].T
    w0c = W0[:, _NIN:].T
    p0 = jnp.stack([b0, g0, be0])
    w1x = W1[:, :_NHID].T
    w1c = W1[:, _NHID:].T
    p1 = jnp.stack([b1, g1, be1])
    whead = jnp.concatenate(
        [padw(Wm), padw(Wv), padw(Wlm), padw(Wrm), padw(Wlv), padw(Wrv),
         jnp.zeros((_NHID, 128 - 6 * _L), f32)], axis=1)
    hb = jnp.concatenate([padv(bm), padv(bv)], axis=0)
    attab = jnp.concatenate([padv(attm), padv(attv)], axis=0)
    eps1p = jnp.asarray(_EPS1P)
    tabs, z, rsum, attp = _encode(
        x, bif, w0x, w0c, p0, w1x, w1c, p1, whead, hb, attab, eps1p)

    edges = edge_index.reshape(2, _NW, _NCHUNK, _CB)
    out = _gat_call(edges, tabs, attp)

    gb = jnp.concatenate([padv(biasm), padv(biasv)], axis=0)
    eps2p = jnp.asarray(_EPS2P)
    wd0z = jnp.concatenate(
        [jnp.pad(Wd0[:, :_NOUT].T, ((0, _L - _NOUT), (0, 0))),
         jnp.pad(Wd0[:, _NOUT:2 * _NOUT].T, ((0, _L - _NOUT), (0, 0)))],
        axis=0)
    wd0c = Wd0[:, 2 * _NOUT:].T
    pd = jnp.stack([bd0, gd0, bed0])
    ps, pr, pdo = _decode(out, z, rsum, bif, gb, eps2p, wd0z, wd0c, pd,
                          Wscale.T, bscale.reshape(1, -1), Wdrop.T,
                          bdrop.reshape(1, -1))
    return (ps, jnp.exp(px_r), pr, pdo)


# speedup vs baseline: 1.0608x; 1.0608x over previous
import functools

import jax
import jax.numpy as jnp
from jax import lax
from jax.experimental import pallas as pl
from jax.experimental.pallas import tpu as pltpu
from jax.experimental.pallas import tpu_sc as plsc

_N = 10000
_E = 320000
_NIN = 128
_NHID = 128
_NOUT = 10
_VAR_EPS = 1e-4
_L = 16
_NC = 2
_NS = 16
_NW = _NC * _NS
_CB = 80
_NCHUNK = 125
_NPAIR = (_NCHUNK - 1) // 2
_EPT = _CB * _NCHUNK
_RPT = 640
_SPAD = _RPT * _NS
_NPAD = 16000
_HP = lax.Precision.HIGHEST

import numpy as _np


def _tf_rounds(x0, x1, rots):
    for r in rots:
        x0 = (x0 + x1).astype(_np.uint32)
        x1 = ((x1 << _np.uint32(r)) | (x1 >> _np.uint32(32 - r))).astype(
            _np.uint32)
        x1 = x1 ^ x0
    return x0, x1


def _threefry2x32(k1, k2, x0, x1):
    r1 = (13, 15, 26, 6)
    r2 = (17, 29, 16, 24)
    ks0 = _np.uint32(k1)
    ks1 = _np.uint32(k2)
    ks2 = ks0 ^ ks1 ^ _np.uint32(0x1BD11BDA)
    u = _np.uint32
    x0 = (x0 + ks0).astype(u)
    x1 = (x1 + ks1).astype(u)
    for i, (ka, kb, rr) in enumerate(((ks1, ks2, r1), (ks2, ks0, r2),
                                      (ks0, ks1, r1), (ks1, ks2, r2),
                                      (ks2, ks0, r1))):
        x0, x1 = _tf_rounds(x0, x1, rr)
        x0 = (x0 + ka).astype(u)
        x1 = (x1 + kb + u(i + 1)).astype(u)
    return x0, x1


def _erfinv32(u):
    w = -_np.log((1.0 - u.astype(_np.float64)) * (1.0 + u))
    p = _np.where(
        w < 5.0,
        _np.polyval([2.81022636e-08, 3.43273939e-07, -3.5233877e-06,
                     -4.39150654e-06, 0.00021858087, -0.00125372503,
                     -0.00417768164, 0.246640727, 1.50140941], w - 2.5),
        _np.polyval([-0.000200214257, 0.000100950558, 0.00134934322,
                     -0.00367342844, 0.00573950773, -0.0076224613,
                     0.00943887047, 1.00167406, 2.83297682],
                    _np.sqrt(_np.maximum(w, 5.0)) - 3.0))
    return (p * u).astype(_np.float32)


def _const_normal(seed):
    cnt = _np.arange(_N * _NOUT, dtype=_np.uint32)
    b1, b2 = _threefry2x32(0, seed, _np.zeros_like(cnt), cnt)
    bits = b1 ^ b2
    fb = (bits >> _np.uint32(9)) | _np.float32(1.0).view(_np.uint32)
    floats = fb.view(_np.float32) - _np.float32(1.0)
    lo = _np.float32(_np.nextafter(_np.float32(-1.0), _np.float32(0.0)))
    hi = _np.float32(1.0)
    uni = _np.maximum(lo, (floats * (hi - lo) + lo).astype(_np.float32))
    vals = (_np.float32(_np.sqrt(2.0)) * _erfinv32(uni)).astype(_np.float32)
    return _np.pad(vals.reshape(_N, _NOUT), ((0, 0), (0, _L - _NOUT)))


_EPS1P = _const_normal(1)
_EPS2P = _const_normal(2)


_GD = lax.GatherDimensionNumbers(
    offset_dims=(), collapsed_slice_dims=(0,), start_index_map=(0,))


def _lanesum(t):
    lanes = lax.iota(jnp.int32, _L)
    for k in (8, 4, 2, 1):
        t = t + lax.gather(t, (lanes ^ k)[:, None], _GD, slice_sizes=(1,),
                           mode=lax.GatherScatterMode.PROMISE_IN_BOUNDS)
    return t


def _bn(h, g, b):
    m = jnp.mean(h, axis=0, keepdims=True)
    v = jnp.mean((h - m) ** 2, axis=0, keepdims=True)
    return (h - m) * lax.rsqrt(v + 1e-5) * g + b


_BLK = 2000
_G = _N // _BLK


def _rowspec(width):
    return pl.BlockSpec((_BLK, width), lambda i: (i, 0))


def _fullspec(shape):
    return pl.BlockSpec(shape, lambda i: (0, 0))


def _stats(st_ref):
    m = st_ref[0:1, :] * (1.0 / _N)
    v = st_ref[1:2, :] * (1.0 / _N) - m * m
    return m, lax.rsqrt(v + 1e-5)


def _enc_a_body(x_ref, bif_ref, w0x_ref, w0c_ref, p0_ref,
                hpre_ref, rsum_ref, st_ref, acc):
    i = pl.program_id(0)

    @pl.when(i == 0)
    def _():
        acc[...] = jnp.zeros_like(acc)

    x = x_ref[...]
    xl = jnp.log1p(x)
    sel = jnp.where(bif_ref[...] == 0, w0c_ref[0:1, :], w0c_ref[1:2, :])
    hp = jnp.dot(xl, w0x_ref[...], precision=_HP) + sel + p0_ref[0:1, :]
    hpre_ref[...] = hp
    rsum_ref[...] = jnp.broadcast_to(
        jnp.sum(x, axis=1, keepdims=True), (_BLK, _L))
    acc[0:1, :] = acc[0:1, :] + jnp.sum(hp, axis=0, keepdims=True)
    acc[1:2, :] = acc[1:2, :] + jnp.sum(hp * hp, axis=0, keepdims=True)

    @pl.when(i == _G - 1)
    def _():
        st_ref[...] = acc[...]


def _enc_b_body(hpre_ref, bif_ref, st0_ref, p0_ref, w1x_ref, w1c_ref, p1_ref,
                qpre_ref, st_ref, acc):
    i = pl.program_id(0)

    @pl.when(i == 0)
    def _():
        acc[...] = jnp.zeros_like(acc)

    m0, r0 = _stats(st0_ref)
    h = jax.nn.relu((hpre_ref[...] - m0) * r0 * p0_ref[1:2, :]
                    + p0_ref[2:3, :])
    sel = jnp.where(bif_ref[...] == 0, w1c_ref[0:1, :], w1c_ref[1:2, :])
    qp = jnp.dot(h, w1x_ref[...], precision=_HP) + sel + p1_ref[0:1, :]
    qpre_ref[...] = qp
    acc[0:1, :] = acc[0:1, :] + jnp.sum(qp, axis=0, keepdims=True)
    acc[1:2, :] = acc[1:2, :] + jnp.sum(qp * qp, axis=0, keepdims=True)

    @pl.when(i == _G - 1)
    def _():
        st_ref[...] = acc[...]


def _enc_c_body(qpre_ref, st1_ref, p1_ref, whead_ref, hb_ref, attab_ref,
                eps1_ref, tabs_ref, z_ref, attp_ref, acc):
    i = pl.program_id(0)

    @pl.when(i == 0)
    def _():
        acc[...] = jnp.zeros_like(acc)

    m1, r1 = _stats(st1_ref)
    q = jax.nn.relu((qpre_ref[...] - m1) * r1 * p1_ref[1:2, :]
                    + p1_ref[2:3, :])
    heads = jnp.dot(q, whead_ref[...], precision=_HP)
    col = lax.broadcasted_iota(jnp.int32, (1, 128), 1)
    t4 = jnp.concatenate(
        [heads[:, 32:96], jnp.zeros((_BLK, 64), jnp.float32)], axis=1)
    tabs_ref[...] = jnp.where((col == 10) | (col == 42), 1.0, t4)
    qm = heads[:, 0:16] + hb_ref[0:1, :]
    qv = jnp.exp(heads[:, 16:32] + hb_ref[1:2, :]) + _VAR_EPS
    z_ref[...] = qm + jnp.sqrt(qv) * eps1_ref[...]
    acc[0:1, :] = jnp.maximum(
        acc[0:1, :], jnp.max(jnp.abs(heads), axis=0, keepdims=True))

    @pl.when(i == _G - 1)
    def _():
        cm = acc[0:1, :]
        am = jnp.abs(attab_ref[0:1, :])
        av = jnp.abs(attab_ref[1:2, :])
        bm = jnp.sum((cm[:, 32:48] + cm[:, 48:64]) * am)
        bv = jnp.sum((cm[:, 64:80] + cm[:, 80:96]) * av)
        attw = jnp.concatenate(
            [attab_ref[...], jnp.zeros((2, 112), jnp.float32)], axis=1)
        rowid = lax.broadcasted_iota(jnp.int32, (8, 128), 0)
        bnds = jnp.where(rowid < 3, bm, bv)
        attp_ref[...] = jnp.where(
            rowid == 0, attw[0:1, :],
            jnp.where(rowid == 1, attw[1:2, :],
                      jnp.where(rowid < 4, bnds, 0.0)))


def _encode(x, bif, w0x, w0c, p0, w1x, w1c, p1, whead, hb, attab, eps1p):
    f32 = jnp.float32
    sds = jax.ShapeDtypeStruct
    hpre, rsum, st0 = pl.pallas_call(
        _enc_a_body,
        grid=(_G,),
        in_specs=[_rowspec(128), _rowspec(1), _fullspec((128, 128)),
                  _fullspec((2, 128)), _fullspec((3, 128))],
        out_specs=[_rowspec(128), _rowspec(_L), _fullspec((8, 128))],
        out_shape=[sds((_N, 128), f32), sds((_N, _L), f32),
                   sds((8, 128), f32)],
        scratch_shapes=[pltpu.VMEM((8, 128), f32)],
    )(x, bif, w0x, w0c, p0)
    qpre, st1 = pl.pallas_call(
        _enc_b_body,
        grid=(_G,),
        in_specs=[_rowspec(128), _rowspec(1), _fullspec((8, 128)),
                  _fullspec((3, 128)), _fullspec((128, 128)),
                  _fullspec((2, 128)), _fullspec((3, 128))],
        out_specs=[_rowspec(128), _fullspec((8, 128))],
        out_shape=[sds((_N, 128), f32), sds((8, 128), f32)],
        scratch_shapes=[pltpu.VMEM((8, 128), f32)],
    )(hpre, bif, st0, p0, w1x, w1c, p1)
    tabs, z, attp = pl.pallas_call(
        _enc_c_body,
        grid=(_G,),
        in_specs=[_rowspec(128), _fullspec((8, 128)), _fullspec((3, 128)),
                  _fullspec((128, 128)), _fullspec((2, 16)),
                  _fullspec((2, 16)), _rowspec(_L)],
        out_specs=[_rowspec(128), _rowspec(_L), _fullspec((8, 128))],
        out_shape=[sds((_SPAD, 128), f32), sds((_N, _L), f32),
                   sds((8, 128), f32)],
        scratch_shapes=[pltpu.VMEM((8, 128), f32)],
    )(qpre, st1, p1, whead, hb, attab, eps1p)
    return tabs, z, rsum, attp


def _gat_body(edge_hbm, xt_hbm, attp_hbm, out_hbm,
              src_v, dst_v, ra0, ra1, ra2, ra3, rb0, rb1, rb2, rb3,
              zbuf, attp_v, xlm_sh, xrm_sh, xlv_sh, xrv_sh, accm, accv, sem):
    c = lax.axis_index("c")
    s = lax.axis_index("s")
    wid = c * _NS + s
    pltpu.sync_copy(attp_hbm, attp_v)
    attm_v = attp_v[0, 0:16]
    attv_v = attp_v[1, 0:16]
    bm_v = attp_v[2, 0:16]
    bv_v = attp_v[3, 0:16]
    row0 = s * _RPT
    st = pl.ds(row0, _RPT)
    pltpu.sync_copy(xt_hbm.at[st, pl.ds(0, 16)], xlm_sh.at[st])
    pltpu.sync_copy(xt_hbm.at[st, pl.ds(16, 16)], xrm_sh.at[st])
    pltpu.sync_copy(xt_hbm.at[st, pl.ds(32, 16)], xlv_sh.at[st])
    pltpu.sync_copy(xt_hbm.at[st, pl.ds(48, 16)], xrv_sh.at[st])

    def _z(i, carry):
        zbuf[i, :] = jnp.zeros((_L,), jnp.float32)
        return carry

    lax.fori_loop(0, _RPT, _z, 0)
    pltpu.sync_copy(zbuf, accm.at[st])
    pltpu.sync_copy(zbuf, accv.at[st])
    pltpu.sync_copy(edge_hbm.at[0, wid], src_v)
    pltpu.sync_copy(edge_hbm.at[1, wid], dst_v)
    plsc.subcore_barrier()

    def _issue(j, rl_m, rr_m, rl_v, rr_v):
        sidx = src_v.at[j]
        didx = dst_v.at[j]
        pltpu.async_copy(xlm_sh.at[sidx], rl_m, sem)
        pltpu.async_copy(xrm_sh.at[didx], rr_m, sem)
        pltpu.async_copy(xlv_sh.at[sidx], rl_v, sem)
        pltpu.async_copy(xrv_sh.at[didx], rr_v, sem)

    def _drain(*bufs):
        for b in bufs:
            pltpu.make_async_copy(
                xt_hbm.at[pl.ds(0, _CB), pl.ds(0, 16)], b, sem).wait()

    def _compute(rl_m, rr_m, rl_v, rr_v):
        @plsc.parallel_loop(0, _CB, 1, unroll=4)
        def _(k):
            a_m = rl_m[k, :]
            b_m = rr_m[k, :]
            mm = a_m + b_m
            lrm = jnp.maximum(mm, mm * 0.2)
            eem = jnp.exp(_lanesum(lrm * attm_v) - bm_v)
            rl_m[k, :] = a_m * eem
            a_v = rl_v[k, :]
            b_v = rr_v[k, :]
            mv = a_v + b_v
            lrv = jnp.maximum(mv, mv * 0.2)
            eev = jnp.exp(_lanesum(lrv * attv_v) - bv_v)
            rl_v[k, :] = a_v * eev

    def _scatter(j, rl_m, rl_v):
        didx = dst_v.at[j]
        pltpu.sync_copy(rl_m, accm.at[didx], add=True)
        pltpu.sync_copy(rl_v, accv.at[didx], add=True)

    _issue(0, ra0, ra1, ra2, ra3)

    def _pair(p, carry):
        j0 = 2 * p
        _issue(j0 + 1, rb0, rb1, rb2, rb3)
        _drain(ra0, ra1, ra2, ra3)
        _compute(ra0, ra1, ra2, ra3)
        _scatter(j0, ra0, ra2)
        _issue(j0 + 2, ra0, ra1, ra2, ra3)
        _drain(rb0, rb1, rb2, rb3)
        _compute(rb0, rb1, rb2, rb3)
        _scatter(j0 + 1, rb0, rb2)
        return carry

    lax.fori_loop(0, _NPAIR, _pair, 0)
    _drain(ra0, ra1, ra2, ra3)
    _compute(ra0, ra1, ra2, ra3)
    _scatter(_NCHUNK - 1, ra0, ra2)
    plsc.subcore_barrier()
    out_row = c * _NPAD + row0
    pltpu.sync_copy(accm.at[st], out_hbm.at[pl.ds(out_row, _RPT), pl.ds(0, 16)])
    pltpu.sync_copy(accv.at[st], out_hbm.at[pl.ds(out_row, _RPT), pl.ds(16, 16)])


def _gat_call(edges, tabs, attp):
    f32 = jnp.float32
    i32 = jnp.int32
    run = functools.partial(
        pl.kernel,
        mesh=plsc.VectorSubcoreMesh(core_axis_name="c", subcore_axis_name="s"),
        compiler_params=pltpu.CompilerParams(use_tc_tiling_on_sc=False),
        out_type=jax.ShapeDtypeStruct((_NC * _NPAD, 128), f32),
        scratch_types=(
            [pltpu.VMEM((_NCHUNK, _CB), i32),
             pltpu.VMEM((_NCHUNK, _CB), i32)]
            + [pltpu.VMEM((_CB, _L), f32) for _ in range(8)]
            + [pltpu.VMEM((_RPT, _L), f32),
               pltpu.VMEM((8, 128), f32),
               pltpu.VMEM_SHARED((_SPAD, _L), f32),
               pltpu.VMEM_SHARED((_SPAD, _L), f32),
               pltpu.VMEM_SHARED((_SPAD, _L), f32),
               pltpu.VMEM_SHARED((_SPAD, _L), f32),
               pltpu.VMEM_SHARED((_SPAD, _L), f32),
               pltpu.VMEM_SHARED((_SPAD, _L), f32),
               pltpu.SemaphoreType.DMA]
        ),
    )(_gat_body)
    return run(edges, tabs, attp)


def _dec_a_body(lo_ref, hi_ref, z_ref, bif_ref, gb_ref,
                eps2_ref, wd0z_ref, wd0c_ref, pd_ref,
                hpre_ref, st_ref, acc):
    i = pl.program_id(0)

    @pl.when(i == 0)
    def _():
        acc[...] = jnp.zeros_like(acc)

    lo = lo_ref[...]
    hi = hi_ref[...]
    nm = lo[:, 0:16] + hi[:, 0:16]
    nv = lo[:, 16:32] + hi[:, 16:32]
    qm = nm / (nm[:, 10:11] + 1e-16) + gb_ref[0:1, :]
    qv = jnp.exp(nv / (nv[:, 10:11] + 1e-16) + gb_ref[1:2, :]) + _VAR_EPS
    zg = qm + jnp.sqrt(qv) * eps2_ref[...]
    zall = jnp.concatenate([zg, z_ref[...]], axis=1)
    seld = jnp.where(bif_ref[...] == 0, wd0c_ref[0:1, :], wd0c_ref[1:2, :])
    hp = jnp.dot(zall, wd0z_ref[...], precision=_HP) + seld + pd_ref[0:1, :]
    hpre_ref[...] = hp
    acc[0:1, :] = acc[0:1, :] + jnp.sum(hp, axis=0, keepdims=True)
    acc[1:2, :] = acc[1:2, :] + jnp.sum(hp * hp, axis=0, keepdims=True)

    @pl.when(i == _G - 1)
    def _():
        st_ref[...] = acc[...]


def _dec_b_body(hpre_ref, std_ref, pd_ref, wsc_ref, bsc_ref, wdr_ref,
                bdr_ref, rsum_ref, ps_ref, pr_ref, pdo_ref):
    md, rd = _stats(std_ref)
    hd = jax.nn.relu((hpre_ref[...] - md) * rd * pd_ref[1:2, :]
                     + pd_ref[2:3, :])
    psl = jnp.dot(hd, wsc_ref[...], precision=_HP) + bsc_ref[...]
    psl = psl - jnp.max(psl, axis=1, keepdims=True)
    epl = jnp.exp(psl)
    ps = epl / jnp.sum(epl, axis=1, keepdims=True)
    ps_ref[...] = ps
    pr_ref[...] = rsum_ref[:, 0:1] * ps
    pdo_ref[...] = jnp.dot(hd, wdr_ref[...], precision=_HP) + bdr_ref[...]


def _decode(out, z, rsum, bif, gb, eps2p, wd0z, wd0c, pd, wsc, bsc,
            wdr, bdr):
    f32 = jnp.float32
    sds = jax.ShapeDtypeStruct
    off = _NPAD // _BLK
    hispec = pl.BlockSpec((_BLK, 128), lambda i: (i + off, 0))
    hpre, std = pl.pallas_call(
        _dec_a_body,
        grid=(_G,),
        in_specs=[_rowspec(128), hispec,
                  _rowspec(_L), _rowspec(1), _fullspec((2, 16)),
                  _rowspec(_L), _fullspec((32, 128)), _fullspec((2, 128)),
                  _fullspec((3, 128))],
        out_specs=[_rowspec(128), _fullspec((8, 128))],
        out_shape=[sds((_N, 128), f32), sds((8, 128), f32)],
        scratch_shapes=[pltpu.VMEM((8, 128), f32)],
    )(out, out, z, bif, gb, eps2p, wd0z, wd0c, pd)
    return pl.pallas_call(
        _dec_b_body,
        grid=(_G,),
        in_specs=[_rowspec(128), _fullspec((8, 128)), _fullspec((3, 128)),
                  _fullspec((128, 128)), _fullspec((1, 128)),
                  _fullspec((128, 128)), _fullspec((1, 128)), _rowspec(_L)],
        out_specs=[_rowspec(128), _rowspec(128), _rowspec(128)],
        out_shape=[sds((_N, 128), f32), sds((_N, 128), f32),
                   sds((_N, 128), f32)],
    )(hpre, std, pd, wsc, bsc, wdr, bdr, rsum)


def kernel(x, batch_index, edge_index, W0, b0, g0, be0, W1, b1, g1, be1, Wm,
           bm, Wv, bv, Wlm, Wrm, attm, biasm, Wlv, Wrv, attv, biasv, Wd0, bd0,
           gd0, bed0, Wscale, bscale, Wdrop, bdrop, px_r):
    f32 = jnp.float32

    def padw(w):
        return jnp.pad(w.T, ((0, 0), (0, _L - _NOUT)))

    def padv(v):
        return jnp.pad(v, (0, _L - _NOUT)).reshape(1, _L)

    bif = batch_index
    w0x = W0[:, :_NIN].T
    w0c = W0[:, _NIN:].T
    p0 = jnp.stack([b0, g0, be0])
    w1x = W1[:, :_NHID].T
    w1c = W1[:, _NHID:].T
    p1 = jnp.stack([b1, g1, be1])
    whead = jnp.concatenate(
        [padw(Wm), padw(Wv), padw(Wlm), padw(Wrm), padw(Wlv), padw(Wrv),
         jnp.zeros((_NHID, 128 - 6 * _L), f32)], axis=1)
    hb = jnp.concatenate([padv(bm), padv(bv)], axis=0)
    attab = jnp.concatenate([padv(attm), padv(attv)], axis=0)
    eps1p = jnp.asarray(_EPS1P)
    tabs, z, rsum, attp = _encode(
        x, bif, w0x, w0c, p0, w1x, w1c, p1, whead, hb, attab, eps1p)

    edges = edge_index.reshape(2, _NW, _NCHUNK, _CB)
    out = _gat_call(edges, tabs, attp)

    gb = jnp.concatenate([padv(biasm), padv(biasv)], axis=0)
    eps2p = jnp.asarray(_EPS2P)
    wd0z = jnp.concatenate(
        [jnp.pad(Wd0[:, :_NOUT].T, ((0, _L - _NOUT), (0, 0))),
         jnp.pad(Wd0[:, _NOUT:2 * _NOUT].T, ((0, _L - _NOUT), (0, 0)))],
        axis=0)
    wd0c = Wd0[:, 2 * _NOUT:].T
    pd = jnp.stack([bd0, gd0, bed0])
    ps, pr, pdo = _decode(out, z, rsum, bif, gb, eps2p, wd0z, wd0c, pd,
                          Wscale.T, bscale.reshape(1, -1), Wdrop.T,
                          bdrop.reshape(1, -1))
    return (ps, jnp.exp(px_r), pr, pdo)
